# baseline (speedup 1.0000x reference)
import jax
import jax.numpy as jnp
from jax.experimental import pallas as pl
from jax.experimental.pallas import tpu as pltpu

_OUT_DIM = 3
_IN_DIM = 6
_LANE_TILE = 131072


def _mlp_t_kernel(x_ref, wt_ref, b_ref, o_ref):
    h = jnp.dot(wt_ref[...], x_ref[...],
                preferred_element_type=jnp.float32) + b_ref[...]
    o_ref[...] = 0.5 * jnp.tanh(h) + 0.5


def kernel(x, w_pad, b_pad):
    B = x.shape[0]
    wt = 0.5 * w_pad[:_IN_DIM, :_OUT_DIM].T
    bt = 0.5 * b_pad[:1, :_OUT_DIM].T
    xt = x.T

    tile = _LANE_TILE if B % _LANE_TILE == 0 else 1 << 13
    B_pad = pl.cdiv(B, tile) * tile
    if B_pad != B:
        xt = jnp.pad(xt, ((0, 0), (0, B_pad - B)))

    out_t = pl.pallas_call(
        _mlp_t_kernel,
        out_shape=jax.ShapeDtypeStruct((_OUT_DIM, B_pad), jnp.float32),
        grid=(B_pad // tile,),
        in_specs=[
            pl.BlockSpec((_IN_DIM, tile), lambda i: (0, i)),
            pl.BlockSpec((_OUT_DIM, _IN_DIM), lambda i: (0, 0)),
            pl.BlockSpec((_OUT_DIM, 1), lambda i: (0, 0)),
        ],
        out_specs=pl.BlockSpec((_OUT_DIM, tile), lambda i: (0, i)),
        compiler_params=pltpu.CompilerParams(
            dimension_semantics=("parallel",),
        ),
    )(xt, wt, bt)

    return out_t[:, :B].T if B_pad != B else out_t.T


# speedup vs baseline: 1.0814x; 1.0814x over previous
import jax
import jax.numpy as jnp
from jax.experimental import pallas as pl
from jax.experimental.pallas import tpu as pltpu

_OUT_DIM = 3
_IN_DIM = 6
_LANE_TILE = 262144


def _mlp_t_kernel(x_ref, wt_ref, b_ref, o_ref):
    h = jnp.dot(wt_ref[...], x_ref[...],
                preferred_element_type=jnp.float32) + b_ref[...]
    o_ref[...] = 0.5 * jnp.tanh(h) + 0.5


def kernel(x, w_pad, b_pad):
    B = x.shape[0]
    wt = 0.5 * w_pad[:_IN_DIM, :_OUT_DIM].T
    bt = 0.5 * b_pad[:1, :_OUT_DIM].T
    xt = x.T

    tile = _LANE_TILE if B % _LANE_TILE == 0 else 1 << 13
    B_pad = pl.cdiv(B, tile) * tile
    if B_pad != B:
        xt = jnp.pad(xt, ((0, 0), (0, B_pad - B)))

    out_t = pl.pallas_call(
        _mlp_t_kernel,
        out_shape=jax.ShapeDtypeStruct((_OUT_DIM, B_pad), jnp.float32),
        grid=(B_pad // tile,),
        in_specs=[
            pl.BlockSpec((_IN_DIM, tile), lambda i: (0, i)),
            pl.BlockSpec((_OUT_DIM, _IN_DIM), lambda i: (0, 0)),
            pl.BlockSpec((_OUT_DIM, 1), lambda i: (0, 0)),
        ],
        out_specs=pl.BlockSpec((_OUT_DIM, tile), lambda i: (0, i)),
        compiler_params=pltpu.CompilerParams(
            dimension_semantics=("parallel",),
        ),
    )(xt, wt, bt)

    return out_t[:, :B].T if B_pad != B else out_t.T
